# Initial kernel scaffold; baseline (speedup 1.0000x reference)
import jax
import jax.numpy as jnp
from jax.experimental import pallas as pl


def kernel(u, i, user_df, item_df, table):
    raise NotImplementedError("write your pallas kernel here")


# speedup vs baseline: 1.7723x; 1.7723x over previous
import jax
import jax.numpy as jnp
from jax import lax
from jax.experimental import pallas as pl
from jax.experimental.pallas import tpu as pltpu
from jax.experimental.pallas import tpu_sc as plsc

B = 16384
DIM = 32
F = 13
FP = 16
NC = 2
NS = 16
NW = NC * NS
BPW = B // NW
C = 64
NCHUNK = BPW // C


def _row_accum(buf, r, s0, s1, q):
    v0 = buf[r, pl.ds(0, 16)]
    v1 = buf[r, pl.ds(16, 16)]
    n2 = jnp.sum(v0 * v0 + v1 * v1)
    bits = lax.bitcast_convert_type(n2, jnp.int32)
    y = lax.bitcast_convert_type(
        jnp.int32(0x5F3759DF) - lax.shift_right_arithmetic(bits, 1), jnp.float32)
    half = jnp.float32(0.5) * n2
    for _ in range(3):
        y = y * (jnp.float32(1.5) - half * y * y)
    scale = jnp.where(n2 > jnp.float32(1.0),
                      y - jnp.float32(1e-7) * (y * y),
                      jnp.float32(1.0))
    return s0 + scale * v0, s1 + scale * v1, q + scale * scale * n2


def _fm_body(u_hbm, i_hbm, udf_hbm, idf_hbm, tab_hbm, out_hbm,
             uidx, iidx, urows, irows, uflat, iflat, embu, embi, outv,
             sem0, sem1):
    wid = lax.axis_index("s") * NC + lax.axis_index("c")
    base = wid * BPW
    lane = lax.iota(jnp.int32, 16)

    def chunk(c, carry):
        cb = base + c * C
        pltpu.sync_copy(u_hbm.at[pl.ds(cb, C)], uidx)
        pltpu.sync_copy(i_hbm.at[pl.ds(cb, C)], iidx)
        cu = pltpu.async_copy(udf_hbm.at[uidx], urows, sem0)
        ci = pltpu.async_copy(idf_hbm.at[iidx], irows, sem1)
        cu.wait()
        ci.wait()
        for b in range(C):
            uflat[pl.ds(b * FP, FP)] = urows[b]
            iflat[pl.ds(b * FP, FP)] = irows[b]
        gu = pltpu.async_copy(
            tab_hbm.at[plsc.Indices(uflat, ignored_value=-1)], embu, sem0)
        gi = pltpu.async_copy(
            tab_hbm.at[plsc.Indices(iflat, ignored_value=-1)], embi, sem1)
        gu.wait()
        gi.wait()

        def b_body(b, carry2):
            z = jnp.zeros((16,), jnp.float32)
            sa0, sa1, qa = z, z, jnp.float32(0.0)
            sb0, sb1, qb = z, z, jnp.float32(0.0)
            rbase = b * FP
            for f in range(F):
                sa0, sa1, qa = _row_accum(embu, rbase + f, sa0, sa1, qa)
                sb0, sb1, qb = _row_accum(embi, rbase + f, sb0, sb1, qb)
            s0 = sa0 + sb0
            s1 = sa1 + sb1
            ssq = jnp.sum(s0 * s0 + s1 * s1)
            val = jnp.float32(0.5) * (ssq - (qa + qb))
            plsc.store_scatter(outv,
                               [jnp.full((16,), b, jnp.int32)],
                               jnp.full((16,), val, jnp.float32),
                               mask=lane == 0)
            return carry2

        lax.fori_loop(0, C, b_body, 0, unroll=False)

        for j in range(C // 16):
            x = outv[pl.ds(j * 16, 16)]
            z = jnp.exp(-jnp.abs(x))
            d = jnp.float32(1.0) + z
            r = jnp.float32(24.0 / 17.0) - jnp.float32(8.0 / 17.0) * d
            for _ in range(3):
                r = r * (jnp.float32(2.0) - d * r)
            outv[pl.ds(j * 16, 16)] = jnp.where(
                x >= jnp.float32(0.0), r, jnp.float32(1.0) - r)
        pltpu.sync_copy(outv, out_hbm.at[pl.ds(cb, C)])
        return carry

    lax.fori_loop(0, NCHUNK, chunk, 0, unroll=False)


_fm = pl.kernel(
    _fm_body,
    out_type=jax.ShapeDtypeStruct((B,), jnp.float32),
    mesh=plsc.VectorSubcoreMesh(core_axis_name="c", subcore_axis_name="s"),
    scratch_types=[
        pltpu.VMEM((C,), jnp.int32),
        pltpu.VMEM((C,), jnp.int32),
        pltpu.VMEM((C, FP), jnp.int32),
        pltpu.VMEM((C, FP), jnp.int32),
        pltpu.VMEM((C * FP,), jnp.int32),
        pltpu.VMEM((C * FP,), jnp.int32),
        pltpu.VMEM((C * FP, DIM), jnp.float32),
        pltpu.VMEM((C * FP, DIM), jnp.float32),
        pltpu.VMEM((C,), jnp.float32),
        pltpu.SemaphoreType.DMA,
        pltpu.SemaphoreType.DMA,
    ],
    compiler_params=pltpu.CompilerParams(
        needs_layout_passes=False, use_tc_tiling_on_sc=False),
)


def kernel(u, i, user_df, item_df, table):
    u = u.astype(jnp.int32)
    i = i.astype(jnp.int32)
    user_df = user_df.astype(jnp.int32)
    item_df = item_df.astype(jnp.int32)
    table = table.astype(jnp.float32)
    updf = jnp.pad(user_df, ((0, 0), (0, FP - F)), constant_values=-1)
    ipdf = jnp.pad(item_df, ((0, 0), (0, FP - F)), constant_values=-1)
    return _fm(u, i, updf, ipdf, table)


# speedup vs baseline: 1.8814x; 1.0616x over previous
import jax
import jax.numpy as jnp
from jax import lax
from jax.experimental import pallas as pl
from jax.experimental.pallas import tpu as pltpu
from jax.experimental.pallas import tpu_sc as plsc

B = 16384
DIM = 32
F = 13
NC = 2
NS = 16
NW = NC * NS
BPW = B // NW
C = 64
NCHUNK = BPW // C


def _row_accum(buf, r, s0, s1, q):
    v0 = buf[r, pl.ds(0, 16)]
    v1 = buf[r, pl.ds(16, 16)]
    n2 = jnp.sum(v0 * v0 + v1 * v1)
    bits = lax.bitcast_convert_type(n2, jnp.int32)
    y = lax.bitcast_convert_type(
        jnp.int32(0x5F3759DF) - lax.shift_right_arithmetic(bits, 1), jnp.float32)
    half = jnp.float32(0.5) * n2
    for _ in range(3):
        y = y * (jnp.float32(1.5) - half * y * y)
    scale = jnp.where(n2 > jnp.float32(1.0),
                      y - jnp.float32(1e-7) * (y * y),
                      jnp.float32(1.0))
    return s0 + scale * v0, s1 + scale * v1, q + scale * scale * n2


def _fm_body(u_hbm, i_hbm, udf_hbm, idf_hbm, tab_hbm, out_hbm,
             uidx, iidx, udfi, idfi, ufeat, ifeat, embu, embi, outv,
             sem0, sem1):
    wid = lax.axis_index("s") * NC + lax.axis_index("c")
    base = wid * BPW
    lane = lax.iota(jnp.int32, 16)

    def chunk(c, carry):
        cb = base + c * C
        pltpu.sync_copy(u_hbm.at[pl.ds(cb, C)], uidx)
        pltpu.sync_copy(i_hbm.at[pl.ds(cb, C)], iidx)
        for f in range(F):
            for j in range(C // 16):
                src = pl.ds(j * 16, 16)
                dst = pl.ds(f * C + j * 16, 16)
                udfi[dst] = uidx[src] * jnp.int32(F) + jnp.int32(f)
                idfi[dst] = iidx[src] * jnp.int32(F) + jnp.int32(f)
        cu = pltpu.async_copy(udf_hbm.at[udfi], ufeat, sem0)
        ci = pltpu.async_copy(idf_hbm.at[idfi], ifeat, sem1)
        cu.wait()
        ci.wait()
        gu = pltpu.async_copy(tab_hbm.at[ufeat], embu, sem0)
        gi = pltpu.async_copy(tab_hbm.at[ifeat], embi, sem1)
        gu.wait()
        gi.wait()

        def b_body(b, carry2):
            z = jnp.zeros((16,), jnp.float32)
            sa0, sa1, qa = z, z, jnp.float32(0.0)
            sb0, sb1, qb = z, z, jnp.float32(0.0)
            for f in range(F):
                sa0, sa1, qa = _row_accum(embu, f * C + b, sa0, sa1, qa)
                sb0, sb1, qb = _row_accum(embi, f * C + b, sb0, sb1, qb)
            s0 = sa0 + sb0
            s1 = sa1 + sb1
            ssq = jnp.sum(s0 * s0 + s1 * s1)
            val = jnp.float32(0.5) * (ssq - (qa + qb))
            plsc.store_scatter(outv,
                               [jnp.full((16,), b, jnp.int32)],
                               jnp.full((16,), val, jnp.float32),
                               mask=lane == 0)
            return carry2

        lax.fori_loop(0, C, b_body, 0, unroll=False)

        for j in range(C // 16):
            x = outv[pl.ds(j * 16, 16)]
            z = jnp.exp(-jnp.abs(x))
            d = jnp.float32(1.0) + z
            r = jnp.float32(24.0 / 17.0) - jnp.float32(8.0 / 17.0) * d
            for _ in range(3):
                r = r * (jnp.float32(2.0) - d * r)
            outv[pl.ds(j * 16, 16)] = jnp.where(
                x >= jnp.float32(0.0), r, jnp.float32(1.0) - r)
        pltpu.sync_copy(outv, out_hbm.at[pl.ds(cb, C)])
        return carry

    lax.fori_loop(0, NCHUNK, chunk, 0, unroll=False)


_fm = pl.kernel(
    _fm_body,
    out_type=jax.ShapeDtypeStruct((B,), jnp.float32),
    mesh=plsc.VectorSubcoreMesh(core_axis_name="c", subcore_axis_name="s"),
    scratch_types=[
        pltpu.VMEM((C,), jnp.int32),
        pltpu.VMEM((C,), jnp.int32),
        pltpu.VMEM((C * F,), jnp.int32),
        pltpu.VMEM((C * F,), jnp.int32),
        pltpu.VMEM((C * F,), jnp.int32),
        pltpu.VMEM((C * F,), jnp.int32),
        pltpu.VMEM((C * F, DIM), jnp.float32),
        pltpu.VMEM((C * F, DIM), jnp.float32),
        pltpu.VMEM((C,), jnp.float32),
        pltpu.SemaphoreType.DMA,
        pltpu.SemaphoreType.DMA,
    ],
    compiler_params=pltpu.CompilerParams(
        needs_layout_passes=False, use_tc_tiling_on_sc=False),
)


def kernel(u, i, user_df, item_df, table):
    u = u.astype(jnp.int32)
    i = i.astype(jnp.int32)
    udf = user_df.astype(jnp.int32).reshape(-1)
    idf = item_df.astype(jnp.int32).reshape(-1)
    table = table.astype(jnp.float32)
    return _fm(u, i, udf, idf, table)
